# baseline (speedup 1.0000x reference)
import functools

import jax
import jax.numpy as jnp
from jax import lax
from jax.experimental import pallas as pl
from jax.experimental.pallas import tpu as pltpu
from jax.experimental.pallas import tpu_sc as plsc

_LANES = 16
_NEG_BIG = -1e30


def _sc_router_body(x_hbm, gw_hbm, mask_hbm, xrow, gw, mrow):
    core = lax.axis_index("c")
    sub = lax.axis_index("s")

    @pl.when(core == 0)
    def _():
        t = sub
        pltpu.sync_copy(x_hbm.at[t], xrow)
        pltpu.sync_copy(gw_hbm, gw)
        n_e, hidden = gw.shape
        n_chunk = hidden // _LANES

        lane = lax.broadcasted_iota(jnp.int32, (_LANES,), 0)
        lg = jnp.full((_LANES,), _NEG_BIG, jnp.float32)
        for e in range(n_e):
            def body(j, acc):
                return acc + (xrow[pl.ds(j * _LANES, _LANES)]
                              * gw[e, pl.ds(j * _LANES, _LANES)])
            acc = lax.fori_loop(0, n_chunk, body,
                                jnp.zeros((_LANES,), jnp.float32))
            lg = jnp.where(lane == e, jnp.sum(acc), lg)

        valid = lane < n_e
        m = jnp.max(lg)
        p = jnp.where(valid, jnp.exp(lg - m), 0.0)
        p = p / jnp.sum(p)
        p1 = jnp.max(p)
        i1 = jnp.min(jnp.where(p == p1, lane, _LANES))
        p_rest = jnp.where(lane == i1, -1.0, p)
        p2 = jnp.max(p_rest)
        i2 = jnp.min(jnp.where(p_rest == p2, lane, _LANES))
        sel = (lane == i1) | (lane == i2)
        top2 = jnp.where(sel, p, 0.0)
        denom = jnp.broadcast_to(p1 + p2, (_LANES,))
        mrow[...] = top2 / denom
        pltpu.sync_copy(mrow, mask_hbm.at[t])


def _sc_router(xf, gate_w, n_tok):
    mesh = plsc.VectorSubcoreMesh(core_axis_name="c", subcore_axis_name="s")
    n_e, hidden = gate_w.shape
    run = functools.partial(
        pl.kernel, mesh=mesh,
        out_type=jax.ShapeDtypeStruct((n_tok, _LANES), jnp.float32),
        scratch_types=[
            pltpu.VMEM((hidden,), jnp.float32),
            pltpu.VMEM((n_e, hidden), jnp.float32),
            pltpu.VMEM((_LANES,), jnp.float32),
        ],
        compiler_params=pltpu.CompilerParams(needs_layout_passes=False),
    )(_sc_router_body)
    return run(xf, gate_w)


def _moe_kernel(x_ref, mask_in, gate_blk, down_blk, up_blk,
                out_ref, ew_ref, mask_ref):
    e = pl.program_id(0)
    i = pl.program_id(1)

    @pl.when((e == 0) & (i == 0))
    def _prologue():
        mfull = mask_in[...]
        mask_ref[...] = mfull.T
        m1 = jnp.max(mfull, axis=1, keepdims=True)
        ew_ref[...] = jnp.concatenate([m1, 1.0 - m1], axis=1)
        out_ref[...] = jnp.zeros_like(out_ref)

    xf = x_ref[...]
    g = jax.lax.dot_general(xf, gate_blk[0], (((1,), (1,)), ((), ())),
                            preferred_element_type=jnp.float32)
    d = jax.lax.dot_general(xf, down_blk[0], (((1,), (1,)), ((), ())),
                            preferred_element_type=jnp.float32)
    h = (g * jax.nn.sigmoid(g)) * d
    w_e = mask_ref[pl.ds(e, 1), :]
    h = h * w_e.reshape(h.shape[0], 1)
    part = jax.lax.dot_general(h, up_blk[0], (((1,), (1,)), ((), ())),
                               preferred_element_type=jnp.float32)
    out_ref[...] += part


def kernel(x, gate_w, gate_proj, up_proj, down_proj):
    batch, seq, hidden = x.shape
    n_tok = batch * seq
    n_exp, inter, _ = gate_proj.shape
    xf = x.reshape(n_tok, hidden)

    mask16 = _sc_router(xf, gate_w, n_tok)

    tile_i = 896
    n_i = inter // tile_i

    out, ew = pl.pallas_call(
        _moe_kernel,
        grid=(n_exp, n_i),
        in_specs=[
            pl.BlockSpec((n_tok, hidden), lambda e, i: (0, 0)),
            pl.BlockSpec((n_tok, _LANES), lambda e, i: (0, 0)),
            pl.BlockSpec((1, tile_i, hidden), lambda e, i: (e, i, 0)),
            pl.BlockSpec((1, tile_i, hidden), lambda e, i: (e, i, 0)),
            pl.BlockSpec((1, hidden, tile_i), lambda e, i: (e, 0, i)),
        ],
        out_specs=[
            pl.BlockSpec((n_tok, hidden), lambda e, i: (0, 0)),
            pl.BlockSpec((n_tok, 2), lambda e, i: (0, 0)),
        ],
        out_shape=[
            jax.ShapeDtypeStruct((n_tok, hidden), jnp.float32),
            jax.ShapeDtypeStruct((n_tok, 2), jnp.float32),
        ],
        scratch_shapes=[pltpu.VMEM((_LANES, n_tok), jnp.float32)],
        compiler_params=pltpu.CompilerParams(
            dimension_semantics=("arbitrary", "arbitrary")),
    )(xf, mask16, gate_proj, down_proj, up_proj)

    return out, ew


# speedup vs baseline: 1.1821x; 1.1821x over previous
import jax
import jax.numpy as jnp
from jax.experimental import pallas as pl
from jax.experimental.pallas import tpu as pltpu


def _moe_kernel(x_ref, gate_w_ref, gate_blk, down_blk, up_blk,
                out_ref, ew_ref, mask_ref):
    e = pl.program_id(0)
    i = pl.program_id(1)

    @pl.when((e == 0) & (i == 0))
    def _router():
        xf = x_ref[...]
        logits = jax.lax.dot_general(
            xf, gate_w_ref[...], (((1,), (1,)), ((), ())),
            preferred_element_type=jnp.float32,
            precision=jax.lax.Precision.HIGHEST)
        m = jnp.max(logits, axis=-1, keepdims=True)
        p = jnp.exp(logits - m)
        p = p / jnp.sum(p, axis=-1, keepdims=True)
        n_e = p.shape[-1]
        idx = jax.lax.broadcasted_iota(jnp.int32, p.shape, 1)
        p1 = jnp.max(p, axis=-1, keepdims=True)
        i1 = jnp.min(jnp.where(p == p1, idx, n_e), axis=-1, keepdims=True)
        p_rest = jnp.where(idx == i1, -1.0, p)
        p2 = jnp.max(p_rest, axis=-1, keepdims=True)
        i2 = jnp.min(jnp.where(p_rest == p2, idx, n_e), axis=-1, keepdims=True)
        s = p1 + p2
        w1 = p1 / s
        w2 = p2 / s
        mask = (jnp.where(idx == i1, w1, 0.0)
                + jnp.where(idx == i2, w2, 0.0))
        mask_ref[...] = mask.T
        ew_ref[...] = jnp.concatenate([w1, w2], axis=-1)
        out_ref[...] = jnp.zeros_like(out_ref)

    xf = x_ref[...]
    g = jax.lax.dot_general(xf, gate_blk[0], (((1,), (1,)), ((), ())),
                            preferred_element_type=jnp.float32)
    d = jax.lax.dot_general(xf, down_blk[0], (((1,), (1,)), ((), ())),
                            preferred_element_type=jnp.float32)
    h = (g * jax.nn.sigmoid(g)) * d
    w_e = mask_ref[pl.ds(e, 1), :]
    h = h * w_e.reshape(h.shape[0], 1)
    part = jax.lax.dot_general(h, up_blk[0], (((1,), (1,)), ((), ())),
                               preferred_element_type=jnp.float32)
    out_ref[...] += part


def kernel(x, gate_w, gate_proj, up_proj, down_proj):
    batch, seq, hidden = x.shape
    n_tok = batch * seq
    n_exp, inter, _ = gate_proj.shape
    xf = x.reshape(n_tok, hidden)

    tile_i = 1792
    n_i = inter // tile_i

    out, ew = pl.pallas_call(
        _moe_kernel,
        grid=(n_exp, n_i),
        in_specs=[
            pl.BlockSpec((n_tok, hidden), lambda e, i: (0, 0)),
            pl.BlockSpec((n_exp, hidden), lambda e, i: (0, 0)),
            pl.BlockSpec((1, tile_i, hidden), lambda e, i: (e, i, 0)),
            pl.BlockSpec((1, tile_i, hidden), lambda e, i: (e, i, 0)),
            pl.BlockSpec((1, hidden, tile_i), lambda e, i: (e, 0, i)),
        ],
        out_specs=[
            pl.BlockSpec((n_tok, hidden), lambda e, i: (0, 0)),
            pl.BlockSpec((n_tok, 2), lambda e, i: (0, 0)),
        ],
        out_shape=[
            jax.ShapeDtypeStruct((n_tok, hidden), jnp.float32),
            jax.ShapeDtypeStruct((n_tok, 2), jnp.float32),
        ],
        scratch_shapes=[pltpu.VMEM((n_exp, n_tok), jnp.float32)],
        compiler_params=pltpu.CompilerParams(
            dimension_semantics=("arbitrary", "arbitrary"),
            vmem_limit_bytes=114 * 1024 * 1024),
    )(xf, gate_w, gate_proj, down_proj, up_proj)

    return out, ew


# speedup vs baseline: 1.1942x; 1.0103x over previous
import jax
import jax.numpy as jnp
from jax.experimental import pallas as pl
from jax.experimental.pallas import tpu as pltpu


def _moe_kernel(x_ref, gate_w_ref, gate_blk, down_blk, up_blk,
                out_ref, ew_ref, mask_ref):
    e = pl.program_id(0)
    i = pl.program_id(1)

    @pl.when((e == 0) & (i == 0))
    def _router():
        xf = x_ref[...]
        logits = jax.lax.dot_general(
            xf, gate_w_ref[...], (((1,), (1,)), ((), ())),
            preferred_element_type=jnp.float32,
            precision=jax.lax.Precision.HIGHEST)
        m = jnp.max(logits, axis=-1, keepdims=True)
        p = jnp.exp(logits - m)
        p = p / jnp.sum(p, axis=-1, keepdims=True)
        n_e = p.shape[-1]
        idx = jax.lax.broadcasted_iota(jnp.int32, p.shape, 1)
        p1 = jnp.max(p, axis=-1, keepdims=True)
        i1 = jnp.min(jnp.where(p == p1, idx, n_e), axis=-1, keepdims=True)
        p_rest = jnp.where(idx == i1, -1.0, p)
        p2 = jnp.max(p_rest, axis=-1, keepdims=True)
        i2 = jnp.min(jnp.where(p_rest == p2, idx, n_e), axis=-1, keepdims=True)
        s = p1 + p2
        w1 = p1 / s
        w2 = p2 / s
        mask = (jnp.where(idx == i1, w1, 0.0)
                + jnp.where(idx == i2, w2, 0.0))
        mask_ref[...] = mask.T
        ew_ref[...] = jnp.concatenate([w1, w2], axis=-1)
        out_ref[...] = jnp.zeros_like(out_ref)

    xf = x_ref[...]
    g = jax.lax.dot_general(xf, gate_blk[0], (((1,), (1,)), ((), ())),
                            preferred_element_type=jnp.float32)
    d = jax.lax.dot_general(xf, down_blk[0], (((1,), (1,)), ((), ())),
                            preferred_element_type=jnp.float32)
    h = (g * jax.nn.sigmoid(g)) * d
    w_e = mask_ref[pl.ds(e, 1), :]
    h = h * w_e.reshape(h.shape[0], 1)
    part = jax.lax.dot_general(h, up_blk[0], (((1,), (1,)), ((), ())),
                               preferred_element_type=jnp.float32)
    out_ref[...] += part


def kernel(x, gate_w, gate_proj, up_proj, down_proj):
    batch, seq, hidden = x.shape
    n_tok = batch * seq
    n_exp, inter, _ = gate_proj.shape
    xf = x.reshape(n_tok, hidden)

    tile_i = 896
    n_i = inter // tile_i

    out, ew = pl.pallas_call(
        _moe_kernel,
        grid=(n_exp, n_i),
        in_specs=[
            pl.BlockSpec((n_tok, hidden), lambda e, i: (0, 0)),
            pl.BlockSpec((n_exp, hidden), lambda e, i: (0, 0)),
            pl.BlockSpec((1, tile_i, hidden), lambda e, i: (e, i, 0)),
            pl.BlockSpec((1, tile_i, hidden), lambda e, i: (e, i, 0)),
            pl.BlockSpec((1, hidden, tile_i), lambda e, i: (e, 0, i)),
        ],
        out_specs=[
            pl.BlockSpec((n_tok, hidden), lambda e, i: (0, 0)),
            pl.BlockSpec((n_tok, 2), lambda e, i: (0, 0)),
        ],
        out_shape=[
            jax.ShapeDtypeStruct((n_tok, hidden), jnp.float32),
            jax.ShapeDtypeStruct((n_tok, 2), jnp.float32),
        ],
        scratch_shapes=[pltpu.VMEM((n_exp, n_tok), jnp.float32)],
        compiler_params=pltpu.CompilerParams(
            dimension_semantics=("arbitrary", "arbitrary"),
            vmem_limit_bytes=114 * 1024 * 1024),
    )(xf, gate_w, gate_proj, down_proj, up_proj)

    return out, ew


# speedup vs baseline: 1.1952x; 1.0009x over previous
import jax
import jax.numpy as jnp
from jax.experimental import pallas as pl
from jax.experimental.pallas import tpu as pltpu


def _moe_kernel(x_ref, gate_w_ref, gate_blk, down_blk, up_blk,
                out_ref, ew_ref, mask_ref):
    e = pl.program_id(0)
    i = pl.program_id(1)

    @pl.when((e == 0) & (i == 0))
    def _router():
        xf = x_ref[...]
        logits = jax.lax.dot_general(
            xf, gate_w_ref[...], (((1,), (1,)), ((), ())),
            preferred_element_type=jnp.float32)
        m = jnp.max(logits, axis=-1, keepdims=True)
        p = jnp.exp(logits - m)
        p = p / jnp.sum(p, axis=-1, keepdims=True)
        n_e = p.shape[-1]
        idx = jax.lax.broadcasted_iota(jnp.int32, p.shape, 1)
        p1 = jnp.max(p, axis=-1, keepdims=True)
        i1 = jnp.min(jnp.where(p == p1, idx, n_e), axis=-1, keepdims=True)
        p_rest = jnp.where(idx == i1, -1.0, p)
        p2 = jnp.max(p_rest, axis=-1, keepdims=True)
        i2 = jnp.min(jnp.where(p_rest == p2, idx, n_e), axis=-1, keepdims=True)
        s = p1 + p2
        w1 = p1 / s
        w2 = p2 / s
        mask = (jnp.where(idx == i1, w1, 0.0)
                + jnp.where(idx == i2, w2, 0.0))
        mask_ref[...] = mask.T
        ew_ref[...] = jnp.concatenate([w1, w2], axis=-1)
        out_ref[...] = jnp.zeros_like(out_ref)

    xf = x_ref[...]
    g = jax.lax.dot_general(xf, gate_blk[0], (((1,), (1,)), ((), ())),
                            preferred_element_type=jnp.float32)
    d = jax.lax.dot_general(xf, down_blk[0], (((1,), (1,)), ((), ())),
                            preferred_element_type=jnp.float32)
    h = (g * jax.nn.sigmoid(g)) * d
    w_e = mask_ref[pl.ds(e, 1), :]
    h = h * w_e.reshape(h.shape[0], 1)
    part = jax.lax.dot_general(h, up_blk[0], (((1,), (1,)), ((), ())),
                               preferred_element_type=jnp.float32)
    out_ref[...] += part


def kernel(x, gate_w, gate_proj, up_proj, down_proj):
    batch, seq, hidden = x.shape
    n_tok = batch * seq
    n_exp, inter, _ = gate_proj.shape
    xf = x.reshape(n_tok, hidden)

    tile_i = 896
    n_i = inter // tile_i

    out, ew = pl.pallas_call(
        _moe_kernel,
        grid=(n_exp, n_i),
        in_specs=[
            pl.BlockSpec((n_tok, hidden), lambda e, i: (0, 0)),
            pl.BlockSpec((n_exp, hidden), lambda e, i: (0, 0)),
            pl.BlockSpec((1, tile_i, hidden), lambda e, i: (e, i, 0)),
            pl.BlockSpec((1, tile_i, hidden), lambda e, i: (e, i, 0)),
            pl.BlockSpec((1, hidden, tile_i), lambda e, i: (e, 0, i)),
        ],
        out_specs=[
            pl.BlockSpec((n_tok, hidden), lambda e, i: (0, 0)),
            pl.BlockSpec((n_tok, 2), lambda e, i: (0, 0)),
        ],
        out_shape=[
            jax.ShapeDtypeStruct((n_tok, hidden), jnp.float32),
            jax.ShapeDtypeStruct((n_tok, 2), jnp.float32),
        ],
        scratch_shapes=[pltpu.VMEM((n_exp, n_tok), jnp.float32)],
        compiler_params=pltpu.CompilerParams(
            dimension_semantics=("arbitrary", "arbitrary"),
            vmem_limit_bytes=114 * 1024 * 1024),
    )(xf, gate_w, gate_proj, down_proj, up_proj)

    return out, ew
